# Initial kernel scaffold; baseline (speedup 1.0000x reference)
import jax
import jax.numpy as jnp
from jax.experimental import pallas as pl


def kernel(features, weight, nodes, neigh_idx):
    raise NotImplementedError("write your pallas kernel here")


# speedup vs baseline: 4.5544x; 4.5544x over previous
import jax
import jax.numpy as jnp
from jax import lax
from jax.experimental import pallas as pl
from jax.experimental.pallas import tpu as pltpu
from jax.experimental.pallas import tpu_sc as plsc

F = 128
B = 50000
S = 10
NW = 32
C = 32
KCH = 49
BP = NW * KCH * C
BT = 1024
GRID = BP // BT


def _sc_body(feat_hbm, nodes_hbm, neigh_hbm, self_out, nsum_out,
             sidx_v, widx_v, nbuf, sbuf, mbuf, gsem, osem):
    wid = lax.axis_index("s") * 2 + lax.axis_index("c")
    crow0 = wid * KCH

    pltpu.sync_copy(nodes_hbm.at[wid], sidx_v)
    pltpu.sync_copy(neigh_hbm.at[wid], widx_v)

    def chunk(k, carry):
        base = (crow0 + k) * C
        hs = pltpu.async_copy(feat_hbm.at[sidx_v.at[k]], sbuf, gsem)
        hn = [pltpu.async_copy(feat_hbm.at[widx_v.at[j, k]], nbuf.at[j], gsem)
              for j in range(S)]
        hs.wait()
        for h in hn:
            h.wait()

        def row(r, rc):
            for c in range(F // 16):
                acc = nbuf[0, r, pl.ds(c * 16, 16)]
                for j in range(1, S):
                    acc = acc + nbuf[j, r, pl.ds(c * 16, 16)]
                mbuf[r, pl.ds(c * 16, 16)] = acc
            return rc
        lax.fori_loop(0, C, row, 0)

        ho1 = pltpu.async_copy(sbuf, self_out.at[pl.ds(base, C)], osem)
        ho2 = pltpu.async_copy(mbuf, nsum_out.at[pl.ds(base, C)], osem)
        ho1.wait()
        ho2.wait()
        return carry

    lax.fori_loop(0, KCH, chunk, 0)


def _sc_gather(features, nodes3, neighT3):
    f32 = jnp.float32
    kfn = pl.kernel(
        _sc_body,
        out_type=[jax.ShapeDtypeStruct((BP, F), f32),
                  jax.ShapeDtypeStruct((BP, F), f32)],
        mesh=plsc.VectorSubcoreMesh(core_axis_name="c", subcore_axis_name="s"),
        scratch_types=[
            pltpu.VMEM((KCH, C), jnp.int32),
            pltpu.VMEM((S, KCH, C), jnp.int32),
            pltpu.VMEM((S, C, F), f32),
            pltpu.VMEM((C, F), f32),
            pltpu.VMEM((C, F), f32),
            pltpu.SemaphoreType.DMA,
            pltpu.SemaphoreType.DMA,
        ],
    )
    return kfn(features, nodes3, neighT3)


def _tc_body(s_ref, n_ref, w_ref, o_ref):
    ns = n_ref[...] * jnp.float32(1.0 / S)
    comb = jnp.concatenate([s_ref[...], ns], axis=1)
    z = lax.dot_general(w_ref[...], comb, (((1,), (1,)), ((), ())),
                        preferred_element_type=jnp.float32)
    o_ref[...] = jnp.tanh(z) + jnp.tanh(ns).T


def _tc_dense(self_p, nsum_p, weight):
    return pl.pallas_call(
        _tc_body,
        grid=(GRID,),
        in_specs=[
            pl.BlockSpec((BT, F), lambda i: (i, 0)),
            pl.BlockSpec((BT, F), lambda i: (i, 0)),
            pl.BlockSpec((F, 2 * F), lambda i: (0, 0)),
        ],
        out_specs=pl.BlockSpec((F, BT), lambda i: (0, i)),
        out_shape=jax.ShapeDtypeStruct((F, B), jnp.float32),
    )(self_p, nsum_p, weight)


def kernel(features, weight, nodes, neigh_idx):
    nodes3 = jnp.pad(nodes, (0, BP - B)).reshape(NW, KCH, C)
    neighT4 = (jnp.pad(neigh_idx, ((0, BP - B), (0, 0)))
               .T.reshape(S, NW, KCH, C).transpose(1, 0, 2, 3))
    self_p, nsum_p = _sc_gather(features, nodes3, neighT4)
    return _tc_dense(self_p, nsum_p, weight)
